# Initial kernel scaffold; baseline (speedup 1.0000x reference)
import jax
import jax.numpy as jnp
from jax.experimental import pallas as pl


def kernel(x, Wqkv, Wproj, bproj, batch_win_inds, offsets, counts):
    raise NotImplementedError("write your pallas kernel here")


# speedup vs baseline: 57.3296x; 57.3296x over previous
import jax
import jax.numpy as jnp
from jax.experimental import pallas as pl
from jax.experimental.pallas import tpu as pltpu

H = 8
D = 64
B = 256


def _qkv_kernel(x_ref, w_ref, o_ref):
    acc = jnp.dot(x_ref[...], w_ref[...], preferred_element_type=jnp.float32)
    c = acc.shape[1] // 3
    o_ref[:, : 2 * c] = jnp.maximum(acc[:, : 2 * c], 0.0)
    o_ref[:, 2 * c :] = acc[:, 2 * c :]


def _bwd_kernel(wr_c_ref, wr_c1_ref, wc_c1_ref, k_ref, v_ref,
                kvb_ref, sb_ref, kv_s, s_s):
    i = pl.program_id(0)
    first = i == 0
    wt = wr_c_ref[0, :, B - 1 : B]
    wnext = wr_c1_ref[0, :, B - 1 : B]
    wcol = wc_c1_ref[0, :, :]
    prop = jnp.logical_and(jnp.logical_not(first), wt == wnext)
    mask = wcol == wt
    km = jnp.where(mask, k_ref[...], 0.0)
    for h in range(H):
        sl = slice(h * D, (h + 1) * D)
        kh = km[:, sl]
        vh = v_ref[:, sl]
        kv_new = jax.lax.dot_general(
            kh, vh, (((0,), (0,)), ((), ())),
            preferred_element_type=jnp.float32)
        kv_new = kv_new + jnp.where(prop, kv_s[sl, :], 0.0)
        kv_new = jnp.where(first, 0.0, kv_new)
        kv_s[sl, :] = kv_new
        kvb_ref[0, sl, :] = kv_new
        s_new = jnp.sum(kh, axis=0, keepdims=True)
        s_new = s_new + jnp.where(prop, s_s[h : h + 1, :], 0.0)
        s_new = jnp.where(first, 0.0, s_new)
        s_s[h : h + 1, :] = s_new
        sb_ref[0, h : h + 1, :] = s_new


def _fwd_kernel(wr_b_ref, wr_b1_ref, wc_b_ref, qkv_ref, kvb_ref, sb_ref,
                wp_ref, bp_ref, o_ref, kv_s, s_s):
    b = pl.program_id(0)

    @pl.when(b == 0)
    def _():
        kv_s[...] = jnp.zeros_like(kv_s)
        s_s[...] = jnp.zeros_like(s_s)

    wrow = wr_b_ref[0, :, :]
    wcol = wc_b_ref[0, :, :]
    ws = wrow[:, 0:1]
    we = wrow[:, B - 1 : B]
    wsn = wr_b1_ref[0, :, 0:1]
    mask_f = wcol == ws
    mask_e = wcol == we
    mask_n = wcol == wsn
    prop_f = ws == wsn
    mask2 = wcol == wrow
    c = H * D
    ys = []
    for h in range(H):
        sl = slice(h * D, (h + 1) * D)
        qh = qkv_ref[:, h * D : (h + 1) * D]
        kh = qkv_ref[:, c + h * D : c + (h + 1) * D]
        vh = qkv_ref[:, 2 * c + h * D : 2 * c + (h + 1) * D]
        a = jax.lax.dot_general(
            qh, kh, (((1,), (1,)), ((), ())),
            preferred_element_type=jnp.float32)
        a = jnp.where(mask2, a, 0.0)
        yh = jnp.dot(a, vh, preferred_element_type=jnp.float32)
        zh = jnp.sum(a, axis=1, keepdims=True)
        kvf_h = kv_s[sl, :]
        sf_h = s_s[h : h + 1, :]
        qf = jnp.where(mask_f, qh, 0.0)
        qe = jnp.where(mask_e, qh, 0.0)
        yh = yh + jnp.dot(qf, kvf_h, preferred_element_type=jnp.float32)
        yh = yh + jnp.dot(qe, kvb_ref[0, sl, :],
                          preferred_element_type=jnp.float32)
        zh = zh + jnp.sum(qf * sf_h, axis=1, keepdims=True)
        zh = zh + jnp.sum(qe * sb_ref[0, h : h + 1, :], axis=1, keepdims=True)
        ys.append(yh / (zh + 1e-6))
        kn = jnp.where(mask_n, kh, 0.0)
        kv_new = jnp.where(prop_f, kvf_h, 0.0) + jax.lax.dot_general(
            kn, vh, (((0,), (0,)), ((), ())),
            preferred_element_type=jnp.float32)
        s_new = (jnp.where(prop_f, sf_h, 0.0)
                 + jnp.sum(kn, axis=0, keepdims=True))
        kv_s[sl, :] = kv_new
        s_s[h : h + 1, :] = s_new
    y = jnp.concatenate(ys, axis=1)
    o_ref[...] = (jnp.dot(y, wp_ref[...], preferred_element_type=jnp.float32)
                  + bp_ref[0, :])


def kernel(x, Wqkv, Wproj, bproj, batch_win_inds, offsets, counts):
    del offsets, counts
    n, c = x.shape
    nb = n // B
    win = batch_win_inds.astype(jnp.int32)
    win_row = win.reshape(nb, 1, B)
    win_col = win.reshape(nb, B, 1)

    qkv = pl.pallas_call(
        _qkv_kernel,
        grid=(n // 512,),
        in_specs=[
            pl.BlockSpec((512, c), lambda i: (i, 0)),
            pl.BlockSpec((c, 3 * c), lambda i: (0, 0)),
        ],
        out_specs=pl.BlockSpec((512, 3 * c), lambda i: (i, 0)),
        out_shape=jax.ShapeDtypeStruct((n, 3 * c), jnp.float32),
    )(x, Wqkv)

    kvb, sb = pl.pallas_call(
        _bwd_kernel,
        grid=(nb,),
        in_specs=[
            pl.BlockSpec((1, 1, B), lambda i: (nb - 1 - i, 0, 0)),
            pl.BlockSpec((1, 1, B), lambda i: (jnp.minimum(nb - i, nb - 1), 0, 0)),
            pl.BlockSpec((1, B, 1), lambda i: (jnp.minimum(nb - i, nb - 1), 0, 0)),
            pl.BlockSpec((B, c), lambda i: (jnp.minimum(nb - i, nb - 1), 1)),
            pl.BlockSpec((B, c), lambda i: (jnp.minimum(nb - i, nb - 1), 2)),
        ],
        out_specs=[
            pl.BlockSpec((1, c, D), lambda i: (nb - 1 - i, 0, 0)),
            pl.BlockSpec((1, H, D), lambda i: (nb - 1 - i, 0, 0)),
        ],
        out_shape=[
            jax.ShapeDtypeStruct((nb, c, D), jnp.float32),
            jax.ShapeDtypeStruct((nb, H, D), jnp.float32),
        ],
        scratch_shapes=[
            pltpu.VMEM((c, D), jnp.float32),
            pltpu.VMEM((H, D), jnp.float32),
        ],
    )(win_row, win_row, win_col, qkv, qkv)

    out = pl.pallas_call(
        _fwd_kernel,
        grid=(nb,),
        in_specs=[
            pl.BlockSpec((1, 1, B), lambda b: (b, 0, 0)),
            pl.BlockSpec((1, 1, B), lambda b: (jnp.minimum(b + 1, nb - 1), 0, 0)),
            pl.BlockSpec((1, B, 1), lambda b: (b, 0, 0)),
            pl.BlockSpec((B, 3 * c), lambda b: (b, 0)),
            pl.BlockSpec((1, c, D), lambda b: (b, 0, 0)),
            pl.BlockSpec((1, H, D), lambda b: (b, 0, 0)),
            pl.BlockSpec((c, c), lambda b: (0, 0)),
            pl.BlockSpec((1, c), lambda b: (0, 0)),
        ],
        out_specs=pl.BlockSpec((B, c), lambda b: (b, 0)),
        out_shape=jax.ShapeDtypeStruct((n, c), jnp.float32),
        scratch_shapes=[
            pltpu.VMEM((c, D), jnp.float32),
            pltpu.VMEM((H, D), jnp.float32),
        ],
    )(win_row, win_row, win_col, qkv, kvb, sb, Wproj, bproj.reshape(1, c))
    return out


# speedup vs baseline: 85.2308x; 1.4867x over previous
import functools

import jax
import jax.numpy as jnp
from jax.experimental import pallas as pl
from jax.experimental.pallas import tpu as pltpu

H = 8
D = 64
DA = 128
B = 256


def _qkv_kernel(x_ref, w_ref, qk_ref, va_ref):
    acc = jnp.dot(x_ref[...], w_ref[...], preferred_element_type=jnp.float32)
    c = H * D
    qk_ref[...] = jnp.maximum(acc[:, : 2 * c], 0.0)
    rows = acc.shape[0]
    one = jnp.ones((rows, 1), jnp.float32)
    zer = jnp.zeros((rows, DA - D - 1), jnp.float32)
    pieces = []
    for h in range(H):
        pieces += [acc[:, 2 * c + h * D : 2 * c + (h + 1) * D], one, zer]
    va_ref[...] = jnp.concatenate(pieces, axis=1)


def _sweep_kernel(edge, wr_cur_ref, wr_adj_ref, wc_adj_ref, qk_ref, va_ref,
                  kv_ref, kv_s):
    i = pl.program_id(0)
    first = i == 0
    wt = wr_cur_ref[0, :, edge : edge + 1]
    wadj = wr_adj_ref[0, :, edge : edge + 1]
    prop = jnp.logical_and(i > 0, wt == wadj)
    mask = wc_adj_ref[0, :, :] == wt
    km = jnp.where(mask, qk_ref[...], 0.0)
    for h in range(H):
        kv_new = jax.lax.dot_general(
            km[:, h * D : (h + 1) * D], va_ref[:, h * DA : (h + 1) * DA],
            (((0,), (0,)), ((), ())),
            preferred_element_type=jnp.float32)
        kv_new = kv_new + jnp.where(prop, kv_s[h * D : (h + 1) * D, :], 0.0)
        kv_new = jnp.where(first, 0.0, kv_new)
        kv_s[h * D : (h + 1) * D, :] = kv_new
        kv_ref[0, h * D : (h + 1) * D, :] = kv_new


def _attn_kernel(wr_ref, wc_ref, qk_ref, va_ref, kvf_ref, kvb_ref, wp_ref,
                 bp_ref, o_ref):
    wrow = wr_ref[0, :, :]
    wcol = wc_ref[0, :, :]
    ws = wrow[:, 0:1]
    we = wrow[:, B - 1 : B]
    mask2 = wcol == wrow
    c = H * D
    q = qk_ref[:, :c]
    qf = jnp.where(wcol == ws, q, 0.0)
    qe = jnp.where(wcol == we, q, 0.0)
    ys = []
    for h in range(H):
        sl = slice(h * D, (h + 1) * D)
        sla = slice(h * DA, (h + 1) * DA)
        qh = q[:, sl]
        kh = qk_ref[:, c + h * D : c + (h + 1) * D]
        a = jax.lax.dot_general(
            qh, kh, (((1,), (1,)), ((), ())),
            preferred_element_type=jnp.float32)
        a = jnp.where(mask2, a, 0.0)
        yz = jnp.dot(a, va_ref[:, sla], preferred_element_type=jnp.float32)
        yz = yz + jnp.dot(qf[:, sl], kvf_ref[0, sl, :],
                          preferred_element_type=jnp.float32)
        yz = yz + jnp.dot(qe[:, sl], kvb_ref[0, sl, :],
                          preferred_element_type=jnp.float32)
        ys.append(yz[:, 0:D] / (yz[:, D : D + 1] + 1e-6))
    y = jnp.concatenate(ys, axis=1)
    o_ref[...] = (jnp.dot(y, wp_ref[...], preferred_element_type=jnp.float32)
                  + bp_ref[0, :])


def kernel(x, Wqkv, Wproj, bproj, batch_win_inds, offsets, counts):
    del offsets, counts
    n, c = x.shape
    nb = n // B
    win = batch_win_inds.astype(jnp.int32)
    win_row = win.reshape(nb, 1, B)
    win_col = win.reshape(nb, B, 1)

    qk, va = pl.pallas_call(
        _qkv_kernel,
        grid=(n // 512,),
        in_specs=[
            pl.BlockSpec((512, c), lambda i: (i, 0)),
            pl.BlockSpec((c, 3 * c), lambda i: (0, 0)),
        ],
        out_specs=[
            pl.BlockSpec((512, 2 * c), lambda i: (i, 0)),
            pl.BlockSpec((512, 2 * c), lambda i: (i, 0)),
        ],
        out_shape=[
            jax.ShapeDtypeStruct((n, 2 * c), jnp.float32),
            jax.ShapeDtypeStruct((n, 2 * c), jnp.float32),
        ],
        compiler_params=pltpu.CompilerParams(
            dimension_semantics=("parallel",)),
    )(x, Wqkv)

    kvb = pl.pallas_call(
        functools.partial(_sweep_kernel, B - 1),
        grid=(nb,),
        in_specs=[
            pl.BlockSpec((1, 1, B), lambda i: (nb - 1 - i, 0, 0)),
            pl.BlockSpec((1, 1, B), lambda i: (jnp.minimum(nb - i, nb - 1), 0, 0)),
            pl.BlockSpec((1, B, 1), lambda i: (jnp.minimum(nb - i, nb - 1), 0, 0)),
            pl.BlockSpec((B, c), lambda i: (jnp.minimum(nb - i, nb - 1), 1)),
            pl.BlockSpec((B, 2 * c), lambda i: (jnp.minimum(nb - i, nb - 1), 0)),
        ],
        out_specs=pl.BlockSpec((1, c, DA), lambda i: (nb - 1 - i, 0, 0)),
        out_shape=jax.ShapeDtypeStruct((nb, c, DA), jnp.float32),
        scratch_shapes=[pltpu.VMEM((c, DA), jnp.float32)],
    )(win_row, win_row, win_col, qk, va)

    kvf = pl.pallas_call(
        functools.partial(_sweep_kernel, 0),
        grid=(nb,),
        in_specs=[
            pl.BlockSpec((1, 1, B), lambda i: (i, 0, 0)),
            pl.BlockSpec((1, 1, B), lambda i: (jnp.maximum(i - 1, 0), 0, 0)),
            pl.BlockSpec((1, B, 1), lambda i: (jnp.maximum(i - 1, 0), 0, 0)),
            pl.BlockSpec((B, c), lambda i: (jnp.maximum(i - 1, 0), 1)),
            pl.BlockSpec((B, 2 * c), lambda i: (jnp.maximum(i - 1, 0), 0)),
        ],
        out_specs=pl.BlockSpec((1, c, DA), lambda i: (i, 0, 0)),
        out_shape=jax.ShapeDtypeStruct((nb, c, DA), jnp.float32),
        scratch_shapes=[pltpu.VMEM((c, DA), jnp.float32)],
    )(win_row, win_row, win_col, qk, va)

    out = pl.pallas_call(
        _attn_kernel,
        grid=(nb,),
        in_specs=[
            pl.BlockSpec((1, 1, B), lambda b: (b, 0, 0)),
            pl.BlockSpec((1, B, 1), lambda b: (b, 0, 0)),
            pl.BlockSpec((B, 2 * c), lambda b: (b, 0)),
            pl.BlockSpec((B, 2 * c), lambda b: (b, 0)),
            pl.BlockSpec((1, c, DA), lambda b: (b, 0, 0)),
            pl.BlockSpec((1, c, DA), lambda b: (b, 0, 0)),
            pl.BlockSpec((c, c), lambda b: (0, 0)),
            pl.BlockSpec((1, c), lambda b: (0, 0)),
        ],
        out_specs=pl.BlockSpec((B, c), lambda b: (b, 0)),
        out_shape=jax.ShapeDtypeStruct((n, c), jnp.float32),
        compiler_params=pltpu.CompilerParams(
            dimension_semantics=("parallel",)),
    )(win_row, win_col, qk, va, kvf, kvb, Wproj, bproj.reshape(1, c))
    return out


# speedup vs baseline: 95.7787x; 1.1238x over previous
import functools

import jax
import jax.numpy as jnp
from jax.experimental import pallas as pl
from jax.experimental.pallas import tpu as pltpu

H = 8
D = 64
DA = 128
B = 256


def _qkv_kernel(x_ref, w_ref, qk_ref, va_ref):
    acc = jnp.dot(x_ref[...], w_ref[...], preferred_element_type=jnp.float32)
    c = H * D
    qk_ref[...] = jnp.maximum(acc[:, : 2 * c], 0.0).astype(jnp.bfloat16)
    rows = acc.shape[0]
    one = jnp.ones((rows, 1), jnp.bfloat16)
    zer = jnp.zeros((rows, DA - D - 1), jnp.bfloat16)
    vb = acc[:, 2 * c :].astype(jnp.bfloat16)
    pieces = []
    for h in range(H):
        pieces += [vb[:, h * D : (h + 1) * D], one, zer]
    va_ref[...] = jnp.concatenate(pieces, axis=1)


def _sweep_kernel(edge, wr_cur_ref, wr_adj_ref, wc_adj_ref, qk_ref, va_ref,
                  kv_ref, kv_s):
    i = pl.program_id(0)
    first = i == 0
    wt = wr_cur_ref[0, :, edge : edge + 1]
    wadj = wr_adj_ref[0, :, edge : edge + 1]
    prop = jnp.logical_and(i > 0, wt == wadj)
    mask = wc_adj_ref[0, :, :] == wt
    km = jnp.where(mask, qk_ref[...], jnp.bfloat16(0))
    for h in range(H):
        kv_new = jax.lax.dot_general(
            km[:, h * D : (h + 1) * D], va_ref[:, h * DA : (h + 1) * DA],
            (((0,), (0,)), ((), ())),
            preferred_element_type=jnp.float32)
        kv_new = kv_new + jnp.where(prop, kv_s[h * D : (h + 1) * D, :], 0.0)
        kv_new = jnp.where(first, 0.0, kv_new)
        kv_s[h * D : (h + 1) * D, :] = kv_new
        kv_ref[0, h * D : (h + 1) * D, :] = kv_new


def _attn_kernel(wr_ref, wc_ref, qk_ref, va_ref, kvf_ref, kvb_ref, wp_ref,
                 bp_ref, o_ref):
    wrow = wr_ref[0, :, :]
    wcol = wc_ref[0, :, :]
    ws = wrow[:, 0:1]
    we = wrow[:, B - 1 : B]
    mask2 = wcol == wrow
    c = H * D
    q = qk_ref[:, :c]
    qf = jnp.where(wcol == ws, q, jnp.bfloat16(0))
    qe = jnp.where(wcol == we, q, jnp.bfloat16(0))
    ys = []
    for h in range(H):
        sl = slice(h * D, (h + 1) * D)
        sla = slice(h * DA, (h + 1) * DA)
        qh = q[:, sl]
        kh = qk_ref[:, c + h * D : c + (h + 1) * D]
        a = jax.lax.dot_general(
            qh, kh, (((1,), (1,)), ((), ())),
            preferred_element_type=jnp.float32)
        a = jnp.where(mask2, a, 0.0)
        yz = jnp.dot(a, va_ref[:, sla], preferred_element_type=jnp.float32)
        yz = yz + jnp.dot(qf[:, sl], kvf_ref[0, sl, :],
                          preferred_element_type=jnp.float32)
        yz = yz + jnp.dot(qe[:, sl], kvb_ref[0, sl, :],
                          preferred_element_type=jnp.float32)
        ys.append(yz[:, 0:D] / (yz[:, D : D + 1] + 1e-6))
    y = jnp.concatenate(ys, axis=1)
    o_ref[...] = (jnp.dot(y, wp_ref[...], preferred_element_type=jnp.float32)
                  + bp_ref[0, :])


def kernel(x, Wqkv, Wproj, bproj, batch_win_inds, offsets, counts):
    del offsets, counts
    n, c = x.shape
    nb = n // B
    win = batch_win_inds.astype(jnp.int32)
    win_row = win.reshape(nb, 1, B)
    win_col = win.reshape(nb, B, 1)

    qk, va = pl.pallas_call(
        _qkv_kernel,
        grid=(n // 512,),
        in_specs=[
            pl.BlockSpec((512, c), lambda i: (i, 0)),
            pl.BlockSpec((c, 3 * c), lambda i: (0, 0)),
        ],
        out_specs=[
            pl.BlockSpec((512, 2 * c), lambda i: (i, 0)),
            pl.BlockSpec((512, 2 * c), lambda i: (i, 0)),
        ],
        out_shape=[
            jax.ShapeDtypeStruct((n, 2 * c), jnp.bfloat16),
            jax.ShapeDtypeStruct((n, 2 * c), jnp.bfloat16),
        ],
        compiler_params=pltpu.CompilerParams(
            dimension_semantics=("parallel",)),
    )(x, Wqkv)

    kvb = pl.pallas_call(
        functools.partial(_sweep_kernel, B - 1),
        grid=(nb,),
        in_specs=[
            pl.BlockSpec((1, 1, B), lambda i: (nb - 1 - i, 0, 0)),
            pl.BlockSpec((1, 1, B), lambda i: (jnp.minimum(nb - i, nb - 1), 0, 0)),
            pl.BlockSpec((1, B, 1), lambda i: (jnp.minimum(nb - i, nb - 1), 0, 0)),
            pl.BlockSpec((B, c), lambda i: (jnp.minimum(nb - i, nb - 1), 1)),
            pl.BlockSpec((B, 2 * c), lambda i: (jnp.minimum(nb - i, nb - 1), 0)),
        ],
        out_specs=pl.BlockSpec((1, c, DA), lambda i: (nb - 1 - i, 0, 0)),
        out_shape=jax.ShapeDtypeStruct((nb, c, DA), jnp.float32),
        scratch_shapes=[pltpu.VMEM((c, DA), jnp.float32)],
    )(win_row, win_row, win_col, qk, va)

    kvf = pl.pallas_call(
        functools.partial(_sweep_kernel, 0),
        grid=(nb,),
        in_specs=[
            pl.BlockSpec((1, 1, B), lambda i: (i, 0, 0)),
            pl.BlockSpec((1, 1, B), lambda i: (jnp.maximum(i - 1, 0), 0, 0)),
            pl.BlockSpec((1, B, 1), lambda i: (jnp.maximum(i - 1, 0), 0, 0)),
            pl.BlockSpec((B, c), lambda i: (jnp.maximum(i - 1, 0), 1)),
            pl.BlockSpec((B, 2 * c), lambda i: (jnp.maximum(i - 1, 0), 0)),
        ],
        out_specs=pl.BlockSpec((1, c, DA), lambda i: (i, 0, 0)),
        out_shape=jax.ShapeDtypeStruct((nb, c, DA), jnp.float32),
        scratch_shapes=[pltpu.VMEM((c, DA), jnp.float32)],
    )(win_row, win_row, win_col, qk, va)

    out = pl.pallas_call(
        _attn_kernel,
        grid=(nb,),
        in_specs=[
            pl.BlockSpec((1, 1, B), lambda b: (b, 0, 0)),
            pl.BlockSpec((1, B, 1), lambda b: (b, 0, 0)),
            pl.BlockSpec((B, 2 * c), lambda b: (b, 0)),
            pl.BlockSpec((B, 2 * c), lambda b: (b, 0)),
            pl.BlockSpec((1, c, DA), lambda b: (b, 0, 0)),
            pl.BlockSpec((1, c, DA), lambda b: (b, 0, 0)),
            pl.BlockSpec((c, c), lambda b: (0, 0)),
            pl.BlockSpec((1, c), lambda b: (0, 0)),
        ],
        out_specs=pl.BlockSpec((B, c), lambda b: (b, 0)),
        out_shape=jax.ShapeDtypeStruct((n, c), jnp.float32),
        compiler_params=pltpu.CompilerParams(
            dimension_semantics=("parallel",)),
    )(win_row, win_col, qk, va, kvf, kvb, Wproj, bproj.reshape(1, c))
    return out
